# Initial kernel scaffold; baseline (speedup 1.0000x reference)
import jax
import jax.numpy as jnp
from jax.experimental import pallas as pl


def kernel(x, prototypes, sa_weights):
    raise NotImplementedError("write your pallas kernel here")


# speedup vs baseline: 1.1043x; 1.1043x over previous
import functools

import jax
import jax.numpy as jnp
from jax.experimental import pallas as pl
from jax.experimental.pallas import tpu as pltpu

TEMP = 0.2
EPS = 1e-12


def _body(x_ref, pt_ref, sa_ref, out_ref, pn_ref, *, n_k, cp):
    @pl.when(pl.program_id(0) == 0)
    def _():
        pt = pt_ref[...]
        n2 = jnp.sum(pt * pt, axis=0, keepdims=True)
        inv = 1.0 / jnp.maximum(jnp.sqrt(n2), EPS)
        pn_ref[...] = (pt * inv).astype(jnp.bfloat16)

    g, n, d = x_ref.shape
    xb = x_ref[...].reshape(g * n, d)
    n2 = jnp.sum(xb * xb, axis=1, keepdims=True)
    inv = 1.0 / jnp.maximum(jnp.sqrt(n2), EPS)
    xn = (xb * inv).astype(jnp.bfloat16)

    logits = jnp.dot(xn, pn_ref[...], preferred_element_type=jnp.float32)
    m = jnp.max(logits.reshape(g, n, n_k * cp), axis=1)

    sa = sa_ref[...]
    slices = [sa[:, k * cp:(k + 1) * cp] for k in range(n_k)]
    mx = slices[0]
    for k in range(1, n_k):
        mx = jnp.maximum(mx, slices[k])
    es = [jnp.exp(s - mx) for s in slices]
    denom = es[0]
    for k in range(1, n_k):
        denom = denom + es[k]

    acc = jnp.zeros((g, cp), jnp.float32)
    for k in range(n_k):
        w = es[k] * (float(n_k) / denom)
        acc = acc + m[:, k * cp:(k + 1) * cp] * w
    out_ref[...] = acc * (1.0 / TEMP)


def kernel(x, prototypes, sa_weights):
    b, n, d = x.shape
    c, n_k, _ = prototypes.shape
    n_classes = c - 1
    cp = 256
    g = 8

    pt = jnp.transpose(prototypes, (1, 0, 2))
    pt = jnp.pad(pt, ((0, 0), (0, cp - c), (0, 0)))
    pt_t = pt.reshape(n_k * cp, d).T

    sa = jnp.pad(jnp.transpose(sa_weights), ((0, 0), (0, cp - n_classes)))
    sa_km = sa.reshape(1, n_k * cp)

    out = pl.pallas_call(
        functools.partial(_body, n_k=n_k, cp=cp),
        grid=(b // g,),
        in_specs=[
            pl.BlockSpec((g, n, d), lambda i: (i, 0, 0)),
            pl.BlockSpec((d, n_k * cp), lambda i: (0, 0)),
            pl.BlockSpec((1, n_k * cp), lambda i: (0, 0)),
        ],
        out_specs=pl.BlockSpec((g, cp), lambda i: (i, 0)),
        out_shape=jax.ShapeDtypeStruct((b, cp), jnp.float32),
        scratch_shapes=[pltpu.VMEM((d, n_k * cp), jnp.bfloat16)],
    )(x, pt_t, sa_km)
    return out[:, :n_classes]


# speedup vs baseline: 1.1049x; 1.0005x over previous
import functools

import jax
import jax.numpy as jnp
from jax.experimental import pallas as pl
from jax.experimental.pallas import tpu as pltpu

TEMP = 0.2
EPS = 1e-12


def _body(x_ref, pt_ref, sa_ref, s0_ref, out_ref, pn_ref, m_ref, *, n_k, ck):
    i = pl.program_id(0)
    nsteps = pl.num_programs(0)

    @pl.when(i == 0)
    def _():
        p = pt_ref[...]
        n2 = jnp.sum(p * p, axis=1, keepdims=True)
        inv = 1.0 / jnp.maximum(jnp.sqrt(n2), EPS)
        pn_ref[...] = jnp.transpose((p * inv).astype(jnp.bfloat16))

    g, n, d = x_ref.shape
    xb = x_ref[...].reshape(g * n, d)
    n2 = jnp.sum(xb * xb, axis=1, keepdims=True)
    inv = 1.0 / jnp.maximum(jnp.sqrt(n2), EPS)
    xn = (xb * inv).astype(jnp.bfloat16)

    logits = jnp.dot(xn, pn_ref[...], preferred_element_type=jnp.float32)
    m_ref[pl.ds(i * g, g), :] = jnp.max(logits.reshape(g, n, ck), axis=1)

    @pl.when(i == nsteps - 1)
    def _():
        sa = sa_ref[...]
        e = jnp.exp(sa - jnp.max(sa))
        s0 = s0_ref[...]
        me = (m_ref[...] * e).astype(jnp.bfloat16)
        num = jnp.dot(me, s0, preferred_element_type=jnp.float32)
        den = jnp.dot(e.astype(jnp.bfloat16), s0,
                      preferred_element_type=jnp.float32)
        out_ref[...] = num * (float(n_k) / TEMP / jnp.maximum(den, 1e-30))


def kernel(x, prototypes, sa_weights):
    b, n, d = x.shape
    c, n_k, _ = prototypes.shape
    n_classes = c - 1
    ck = 1024
    cp = 256
    g = 8

    pt = jnp.pad(prototypes.reshape(c * n_k, d),
                 ((0, ck - c * n_k), (0, 0)))
    sa = jnp.pad(sa_weights.reshape(1, n_classes * n_k),
                 ((0, 0), (0, ck - n_classes * n_k)), constant_values=-1e30)
    s0 = (jax.lax.broadcasted_iota(jnp.int32, (ck, cp), 0) // n_k
          == jax.lax.broadcasted_iota(jnp.int32, (ck, cp), 1)
          ).astype(jnp.bfloat16)

    out = pl.pallas_call(
        functools.partial(_body, n_k=n_k, ck=ck),
        grid=(b // g,),
        in_specs=[
            pl.BlockSpec((g, n, d), lambda i: (i, 0, 0)),
            pl.BlockSpec((ck, d), lambda i: (0, 0)),
            pl.BlockSpec((1, ck), lambda i: (0, 0)),
            pl.BlockSpec((ck, cp), lambda i: (0, 0)),
        ],
        out_specs=pl.BlockSpec((b, cp), lambda i: (0, 0)),
        out_shape=jax.ShapeDtypeStruct((b, cp), jnp.float32),
        scratch_shapes=[pltpu.VMEM((d, ck), jnp.bfloat16),
                        pltpu.VMEM((b, ck), jnp.float32)],
    )(x, pt, sa, s0)
    return out[:, :n_classes]
